# Initial kernel scaffold; baseline (speedup 1.0000x reference)
import jax
import jax.numpy as jnp
from jax.experimental import pallas as pl


def kernel(x, bridge, up_w, up_b, w1, b1, w2, b2, wid, bid):
    raise NotImplementedError("write your pallas kernel here")


# speedup vs baseline: 1.0377x; 1.0377x over previous
import functools

import jax
import jax.numpy as jnp
from jax import lax
from jax.experimental import pallas as pl
from jax.experimental.pallas import tpu as pltpu


def _leaky(x, slope):
    return jnp.where(x >= 0, x, slope * x)


def _fused_kernel(x_ref, br_ref, w4_ref, bup_ref, w1_ref, b1_ref,
                  w2_ref, b2_ref, wid_ref, bid_ref,
                  o_ref, cpad_ref, y1pad_ref,
                  *, H, W, OH, OW, cin, cup, cout, slope):
    f32 = jnp.float32
    bf16 = jnp.bfloat16
    ccat = cup + br_ref.shape[-1]

    zrow_c = jnp.zeros((1, OW + 2, ccat), bf16)
    zcol_c = jnp.zeros((OH + 2, 1, ccat), bf16)
    cpad_ref[0:1, :, :] = zrow_c
    cpad_ref[OH + 1:OH + 2, :, :] = zrow_c
    cpad_ref[:, 0:1, :] = zcol_c
    cpad_ref[:, OW + 1:OW + 2, :] = zcol_c
    zrow_y = jnp.zeros((1, OW + 2, cout), bf16)
    zcol_y = jnp.zeros((OH + 2, 1, cout), bf16)
    y1pad_ref[0:1, :, :] = zrow_y
    y1pad_ref[OH + 1:OH + 2, :, :] = zrow_y
    y1pad_ref[:, 0:1, :] = zcol_y
    y1pad_ref[:, OW + 1:OW + 2, :] = zcol_y

    x_c = x_ref[0].reshape(cin, H * W)
    up4 = lax.dot_general(x_c, w4_ref[...], (((0,), (0,)), ((), ())),
                          preferred_element_type=f32)
    up4 = up4 + bup_ref[...]
    up = up4.reshape(H, W, 2, 2, cup).transpose(0, 2, 1, 3, 4)
    up = up.reshape(OH, OW, cup).astype(bf16)

    cpad_ref[1:OH + 1, 1:OW + 1, 0:cup] = up
    cpad_ref[1:OH + 1, 1:OW + 1, cup:ccat] = br_ref[0]

    acc = jnp.zeros((OH * OW, cout), f32)
    for k in range(9):
        dy, dx = divmod(k, 3)
        p = cpad_ref[dy:dy + OH, dx:dx + OW, :].reshape(OH * OW, ccat)
        acc = acc + jnp.dot(p, w1_ref[k], preferred_element_type=f32)
    y1 = _leaky(acc + b1_ref[...], slope)
    y1pad_ref[1:OH + 1, 1:OW + 1, :] = y1.reshape(OH, OW, cout).astype(bf16)

    acc2 = jnp.zeros((OH * OW, cout), f32)
    for k in range(9):
        dy, dx = divmod(k, 3)
        p = y1pad_ref[dy:dy + OH, dx:dx + OW, :].reshape(OH * OW, cout)
        acc2 = acc2 + jnp.dot(p, w2_ref[k], preferred_element_type=f32)
    y2 = _leaky(acc2 + b2_ref[...], slope)

    xcat = cpad_ref[1:OH + 1, 1:OW + 1, :].reshape(OH * OW, ccat)
    ident = jnp.dot(xcat, wid_ref[...], preferred_element_type=f32)
    o_ref[0] = (y2 + ident + bid_ref[...]).reshape(OH, OW, cout)


def kernel(x, bridge, up_w, up_b, w1, b1, w2, b2, wid, bid):
    bf16 = jnp.bfloat16
    N, CIN, H, W = x.shape
    Cbr = bridge.shape[1]
    Cup = up_w.shape[1]
    Cout = w1.shape[0]
    Ccat = Cup + Cbr
    OH, OW = 2 * H, 2 * W
    slope = 0.2

    w4 = jnp.transpose(up_w, (0, 2, 3, 1)).reshape(CIN, 4 * Cup).astype(bf16)
    bup = jnp.tile(up_b, 4).reshape(1, 4 * Cup)
    w1r = jnp.transpose(w1, (2, 3, 1, 0)).reshape(9, Ccat, Cout).astype(bf16)
    w2r = jnp.transpose(w2, (2, 3, 1, 0)).reshape(9, Cout, Cout).astype(bf16)
    widr = jnp.transpose(wid[:, :, 0, 0], (1, 0)).astype(bf16)
    b1r = b1.reshape(1, Cout)
    b2r = b2.reshape(1, Cout)
    bidr = bid.reshape(1, Cout)

    xb = x.astype(bf16)
    brb = jnp.transpose(bridge, (0, 2, 3, 1)).astype(bf16)

    kern = functools.partial(_fused_kernel, H=H, W=W, OH=OH, OW=OW,
                             cin=CIN, cup=Cup, cout=Cout, slope=slope)
    out = pl.pallas_call(
        kern,
        out_shape=jax.ShapeDtypeStruct((N, OH, OW, Cout), jnp.float32),
        grid=(N,),
        in_specs=[
            pl.BlockSpec((1, CIN, H, W), lambda n: (n, 0, 0, 0)),
            pl.BlockSpec((1, OH, OW, Cbr), lambda n: (n, 0, 0, 0)),
            pl.BlockSpec((CIN, 4 * Cup), lambda n: (0, 0)),
            pl.BlockSpec((1, 4 * Cup), lambda n: (0, 0)),
            pl.BlockSpec((9, Ccat, Cout), lambda n: (0, 0, 0)),
            pl.BlockSpec((1, Cout), lambda n: (0, 0)),
            pl.BlockSpec((9, Cout, Cout), lambda n: (0, 0, 0)),
            pl.BlockSpec((1, Cout), lambda n: (0, 0)),
            pl.BlockSpec((Ccat, Cout), lambda n: (0, 0)),
            pl.BlockSpec((1, Cout), lambda n: (0, 0)),
        ],
        out_specs=pl.BlockSpec((1, OH, OW, Cout), lambda n: (n, 0, 0, 0)),
        scratch_shapes=[
            pltpu.VMEM((OH + 2, OW + 2, Ccat), bf16),
            pltpu.VMEM((OH + 2, OW + 2, Cout), bf16),
        ],
        compiler_params=pltpu.CompilerParams(
            dimension_semantics=("parallel",),
            vmem_limit_bytes=64 * 1024 * 1024,
        ),
        cost_estimate=pl.CostEstimate(
            flops=2 * N * (H * W * CIN * 4 * Cup
                           + OH * OW * (9 * Ccat * Cout + 9 * Cout * Cout
                                        + Ccat * Cout)),
            transcendentals=0,
            bytes_accessed=2 * N * (CIN * H * W + OH * OW * Cbr)
                           + 4 * N * OH * OW * Cout,
        ),
    )(xb, brb, w4, bup, w1r, b1r, w2r, b2r, widr, bidr)
    return jnp.transpose(out, (0, 3, 1, 2))


# speedup vs baseline: 1.2065x; 1.1627x over previous
import functools

import jax
import jax.numpy as jnp
from jax import lax
from jax.experimental import pallas as pl
from jax.experimental.pallas import tpu as pltpu


def _leaky(x, slope):
    return jnp.where(x >= 0, x, slope * x)


def _shift_right(v, OW):
    return jnp.pad(v, ((0, 0), (1, 0), (0, 0)))[:, :OW, :]


def _shift_left(v, OW):
    return jnp.pad(v, ((0, 0), (0, 1), (0, 0)))[:, 1:OW + 1, :]


def _fused_kernel(x_ref, br_ref, w4_ref, bup_ref, w1_ref, b1_ref,
                  w2_ref, b2_ref, wid_ref, bid_ref,
                  o_ref, b1buf_ref, b2buf_ref,
                  *, H, W, OH, OW, cin, cup, cbr, cout, slope):
    f32 = jnp.float32
    bf16 = jnp.bfloat16
    ccat = cup + cbr

    b1buf_ref[0:1] = jnp.zeros((1, OW, 3 * ccat), bf16)
    b1buf_ref[OH + 1:OH + 2] = jnp.zeros((1, OW, 3 * ccat), bf16)
    b2buf_ref[0:1] = jnp.zeros((1, OW, 3 * cout), bf16)
    b2buf_ref[OH + 1:OH + 2] = jnp.zeros((1, OW, 3 * cout), bf16)

    x_c = x_ref[0].reshape(cin, H * W)
    up4 = lax.dot_general(x_c, w4_ref[...], (((0,), (0,)), ((), ())),
                          preferred_element_type=f32)
    up4 = up4 + bup_ref[...]
    up = up4.reshape(H, W, 2, 2, cup).transpose(0, 2, 1, 3, 4)
    up = up.reshape(OH, OW, cup).astype(bf16)

    br = br_ref[0]

    b1buf_ref[1:OH + 1, :, 0:cup] = _shift_right(up, OW)
    b1buf_ref[1:OH + 1, :, cup:ccat] = _shift_right(br, OW)
    b1buf_ref[1:OH + 1, :, ccat:ccat + cup] = up
    b1buf_ref[1:OH + 1, :, ccat + cup:2 * ccat] = br
    b1buf_ref[1:OH + 1, :, 2 * ccat:2 * ccat + cup] = _shift_left(up, OW)
    b1buf_ref[1:OH + 1, :, 2 * ccat + cup:3 * ccat] = _shift_left(br, OW)

    acc = jnp.zeros((OH * OW, cout), f32)
    for dy in range(3):
        p = b1buf_ref[dy:dy + OH].reshape(OH * OW, 3 * ccat)
        acc = acc + jnp.dot(p, w1_ref[dy], preferred_element_type=f32)
    y1 = _leaky(acc + b1_ref[...], slope)
    y1 = y1.reshape(OH, OW, cout).astype(bf16)

    b2buf_ref[1:OH + 1, :, 0:cout] = _shift_right(y1, OW)
    b2buf_ref[1:OH + 1, :, cout:2 * cout] = y1
    b2buf_ref[1:OH + 1, :, 2 * cout:3 * cout] = _shift_left(y1, OW)

    acc2 = jnp.zeros((OH * OW, cout), f32)
    for dy in range(3):
        p = b2buf_ref[dy:dy + OH].reshape(OH * OW, 3 * cout)
        acc2 = acc2 + jnp.dot(p, w2_ref[dy], preferred_element_type=f32)
    y2 = _leaky(acc2 + b2_ref[...], slope)

    xcat = b1buf_ref[1:OH + 1, :, ccat:2 * ccat].reshape(OH * OW, ccat)
    ident = jnp.dot(xcat, wid_ref[...], preferred_element_type=f32)
    o_ref[0] = (y2 + ident + bid_ref[...]).reshape(OH, OW, cout)


def kernel(x, bridge, up_w, up_b, w1, b1, w2, b2, wid, bid):
    bf16 = jnp.bfloat16
    N, CIN, H, W = x.shape
    Cbr = bridge.shape[1]
    Cup = up_w.shape[1]
    Cout = w1.shape[0]
    Ccat = Cup + Cbr
    OH, OW = 2 * H, 2 * W
    slope = 0.2

    w4 = jnp.transpose(up_w, (0, 2, 3, 1)).reshape(CIN, 4 * Cup).astype(bf16)
    bup = jnp.tile(up_b, 4).reshape(1, 4 * Cup)
    w1r = jnp.transpose(w1, (2, 3, 1, 0)).reshape(3, 3 * Ccat, Cout).astype(bf16)
    w2r = jnp.transpose(w2, (2, 3, 1, 0)).reshape(3, 3 * Cout, Cout).astype(bf16)
    widr = jnp.transpose(wid[:, :, 0, 0], (1, 0)).astype(bf16)
    b1r = b1.reshape(1, Cout)
    b2r = b2.reshape(1, Cout)
    bidr = bid.reshape(1, Cout)

    xb = x.astype(bf16)
    brb = jnp.transpose(bridge, (0, 2, 3, 1)).astype(bf16)

    kern = functools.partial(_fused_kernel, H=H, W=W, OH=OH, OW=OW,
                             cin=CIN, cup=Cup, cbr=Cbr, cout=Cout, slope=slope)
    out = pl.pallas_call(
        kern,
        out_shape=jax.ShapeDtypeStruct((N, OH, OW, Cout), jnp.float32),
        grid=(N,),
        in_specs=[
            pl.BlockSpec((1, CIN, H, W), lambda n: (n, 0, 0, 0)),
            pl.BlockSpec((1, OH, OW, Cbr), lambda n: (n, 0, 0, 0)),
            pl.BlockSpec((CIN, 4 * Cup), lambda n: (0, 0)),
            pl.BlockSpec((1, 4 * Cup), lambda n: (0, 0)),
            pl.BlockSpec((3, 3 * Ccat, Cout), lambda n: (0, 0, 0)),
            pl.BlockSpec((1, Cout), lambda n: (0, 0)),
            pl.BlockSpec((3, 3 * Cout, Cout), lambda n: (0, 0, 0)),
            pl.BlockSpec((1, Cout), lambda n: (0, 0)),
            pl.BlockSpec((Ccat, Cout), lambda n: (0, 0)),
            pl.BlockSpec((1, Cout), lambda n: (0, 0)),
        ],
        out_specs=pl.BlockSpec((1, OH, OW, Cout), lambda n: (n, 0, 0, 0)),
        scratch_shapes=[
            pltpu.VMEM((OH + 2, OW, 3 * Ccat), bf16),
            pltpu.VMEM((OH + 2, OW, 3 * Cout), bf16),
        ],
        compiler_params=pltpu.CompilerParams(
            dimension_semantics=("parallel",),
            vmem_limit_bytes=64 * 1024 * 1024,
        ),
        cost_estimate=pl.CostEstimate(
            flops=2 * N * (H * W * CIN * 4 * Cup
                           + OH * OW * (9 * Ccat * Cout + 9 * Cout * Cout
                                        + Ccat * Cout)),
            transcendentals=0,
            bytes_accessed=2 * N * (CIN * H * W + OH * OW * Cbr)
                           + 4 * N * OH * OW * Cout,
        ),
    )(xb, brb, w4, bup, w1r, b1r, w2r, b2r, widr, bidr)
    return jnp.transpose(out, (0, 3, 1, 2))
